# Initial kernel scaffold; baseline (speedup 1.0000x reference)
import jax
import jax.numpy as jnp
from jax.experimental import pallas as pl


def kernel(features, uni_to_sub, sub_to_uni, num_elements, num_subsets, W_sub0, b_sub0, W_uni0, b_uni0, W_sub1, b_sub1, W_uni1, b_uni1):
    raise NotImplementedError("write your pallas kernel here")


# speedup vs baseline: 56.2598x; 56.2598x over previous
import functools

import jax
import jax.numpy as jnp
from jax import lax
from jax.experimental import pallas as pl
from jax.experimental.pallas import tpu as pltpu
from jax.experimental.pallas import tpu_sc as plsc

N = 10000
NE = 8000
NSUB = 2000
D = 128
E = 320000
NNET = 2

NCORE = 2
NSC = 16
NW = NCORE * NSC

KB = 128
EP = 327680
ROWS = EP // KB
RPW = ROWS // NW
RPW1 = ROWS // NSC

HIST = 10240
ZCH = HIST // NSC
SUBP = 2048
UNIP = 8192

_MESH = plsc.VectorSubcoreMesh(core_axis_name="c", subcore_axis_name="s")


def _deg_body(u2s_r, s2u_r, zeros_in, ones_in, deg1, deg2,
              dbuf, ones_v, hist1, hist2):
    c = lax.axis_index("c")
    s = lax.axis_index("s")
    wid = c * NSC + s
    pltpu.sync_copy(ones_in, ones_v)
    pltpu.sync_copy(zeros_in, hist1.at[pl.ds(s * ZCH, ZCH)])
    pltpu.sync_copy(zeros_in, hist2.at[pl.ds(s * ZCH, ZCH)])
    plsc.subcore_barrier()
    r0 = wid * RPW
    pltpu.sync_copy(u2s_r.at[1, pl.ds(r0, RPW), :], dbuf)

    def ch1(i, carry):
        pltpu.sync_copy(ones_v, hist1.at[dbuf.at[i]], add=True)
        return carry

    lax.fori_loop(0, RPW, ch1, 0)
    pltpu.sync_copy(s2u_r.at[1, pl.ds(r0, RPW), :], dbuf)

    def ch2(i, carry):
        pltpu.sync_copy(ones_v, hist2.at[dbuf.at[i]], add=True)
        return carry

    lax.fori_loop(0, RPW, ch2, 0)
    plsc.subcore_barrier()
    pltpu.sync_copy(hist1.at[pl.ds(s * ZCH, ZCH)],
                    deg1.at[pl.ds(c * HIST + s * ZCH, ZCH)])
    pltpu.sync_copy(hist2.at[pl.ds(s * ZCH, ZCH)],
                    deg2.at[pl.ds(c * HIST + s * ZCH, ZCH)])


_deg_call = functools.partial(
    pl.kernel,
    out_type=(jax.ShapeDtypeStruct((NCORE * HIST,), jnp.float32),
              jax.ShapeDtypeStruct((NCORE * HIST,), jnp.float32)),
    mesh=_MESH,
    scratch_types=[
        pltpu.VMEM((RPW, KB), jnp.int32),
        pltpu.VMEM((KB,), jnp.float32),
        pltpu.VMEM_SHARED((HIST,), jnp.float32),
        pltpu.VMEM_SHARED((HIST,), jnp.float32),
    ],
)(_deg_body)


def _prep_body(deg1_ref, deg2_ref, x_ref, y1_ref, y2_ref, dis1_ref, dis2_ref):
    deg1 = deg1_ref[...]
    deg2 = deg2_ref[...]
    dis1 = lax.rsqrt(deg1[:HIST] + deg1[HIST:] + 1.0)
    dis2 = lax.rsqrt(deg2[:HIST] + deg2[HIST:] + 1.0)
    dis1_ref[...] = dis1
    dis2_ref[...] = dis2
    x = x_ref[...]
    y1_ref[...] = dis1[:N].reshape(N, 1) * x
    y2_ref[...] = dis2[:N].reshape(N, 1) * x


_prep_call = pl.pallas_call(
    _prep_body,
    out_shape=(jax.ShapeDtypeStruct((N, D), jnp.float32),
               jax.ShapeDtypeStruct((N, D), jnp.float32),
               jax.ShapeDtypeStruct((HIST,), jnp.float32),
               jax.ShapeDtypeStruct((HIST,), jnp.float32)),
)


def _scatter_body(y1, y2, u2s_r, s2u_r, zrow_in, accS_out, accU_out,
                  sbuf, dbuf, dloc, rowbuf, accS, accU, sem):
    c = lax.axis_index("c")
    s = lax.axis_index("s")
    wid = c * NSC + s
    pltpu.sync_copy(zrow_in, accS.at[pl.ds(s * 128, 128), :])
    for k in range(UNIP // NSC // 128):
        pltpu.sync_copy(zrow_in, accU.at[pl.ds(s * (UNIP // NSC) + k * 128, 128), :])
    plsc.subcore_barrier()
    r0 = wid * RPW

    def run_list(edges_r, y, acc, in_lo, in_n, dump_base, dump_mask):
        pltpu.sync_copy(edges_r.at[0, pl.ds(r0, RPW), :], sbuf)
        pltpu.sync_copy(edges_r.at[1, pl.ds(r0, RPW), :], dbuf)

        def ch(i, carry):
            cp = pltpu.async_copy(y.at[sbuf.at[i]], rowbuf, sem)

            def loc(j, carry2):
                d = dbuf[i, pl.ds(j * 16, 16)]
                inside = jnp.logical_and(d >= in_lo, d < in_lo + in_n)
                dloc[i, pl.ds(j * 16, 16)] = jnp.where(
                    inside, d - in_lo, dump_base + (d & dump_mask))
                return carry2

            lax.fori_loop(0, KB // 16, loc, 0)
            cp.wait()
            pltpu.sync_copy(rowbuf, acc.at[dloc.at[i]], add=True)
            return carry

        lax.fori_loop(0, RPW, ch, 0)

    run_list(u2s_r, y1, accS, NE, NSUB, NSUB, 31)
    run_list(s2u_r, y2, accU, 0, NE, NE, 127)
    plsc.subcore_barrier()
    pltpu.sync_copy(accS.at[pl.ds(s * 128, 128), :],
                    accS_out.at[c, pl.ds(s * 128, 128), :])
    for k in range(UNIP // NSC // 128):
        o = s * (UNIP // NSC) + k * 128
        pltpu.sync_copy(accU.at[pl.ds(o, 128), :],
                        accU_out.at[c, pl.ds(o, 128), :])


_scatter_call = functools.partial(
    pl.kernel,
    out_type=(jax.ShapeDtypeStruct((NCORE, SUBP, D), jnp.float32),
              jax.ShapeDtypeStruct((NCORE, UNIP, D), jnp.float32)),
    mesh=_MESH,
    scratch_types=[
        pltpu.VMEM((RPW, KB), jnp.int32),
        pltpu.VMEM((RPW, KB), jnp.int32),
        pltpu.VMEM((RPW, KB), jnp.int32),
        pltpu.VMEM((KB, D), jnp.float32),
        pltpu.VMEM_SHARED((SUBP, D), jnp.float32),
        pltpu.VMEM_SHARED((UNIP, D), jnp.float32),
        pltpu.SemaphoreType.DMA,
    ],
)(_scatter_body)


def _dense_body(accS_ref, accU_ref, x_ref, dis1_ref, dis2_ref,
                Ws0_ref, bs0_ref, Wu0_ref, bu0_ref, Ws1_ref,
                t0_ref, t1_ref):
    accS = accS_ref[...]
    accU = accU_ref[...]
    x = x_ref[...]
    dis1 = dis1_ref[...]
    dis2 = dis2_ref[...]
    dS = dis1[NE:N].reshape(NSUB, 1)
    aggS = dS * (accS[0, :NSUB] + accS[1, :NSUB]) + (dS * dS) * x[NE:]
    dU = dis2[:NE].reshape(NE, 1)
    aggU = dU * (accU[0, :NE] + accU[1, :NE]) + (dU * dU) * x[:NE]
    d1 = dis1[:N]
    t_refs = (t0_ref, t1_ref)
    for i in range(NNET):
        S = jnp.dot(aggS, Ws0_ref[i].T, preferred_element_type=jnp.float32) + bs0_ref[i]
        U = jnp.dot(aggU, Wu0_ref[i].T, preferred_element_type=jnp.float32) + bu0_ref[i]
        f = jnp.concatenate([U, S], axis=0)
        f = 1.0 / (1.0 + jnp.exp(-f))
        z = jnp.dot(f, Ws1_ref[i].reshape(D, 1), preferred_element_type=jnp.float32)
        t = z[:, 0] * d1
        t_refs[i][...] = jnp.concatenate([t, jnp.zeros((HIST - N,), jnp.float32)])


_dense_call = pl.pallas_call(
    _dense_body,
    out_shape=(jax.ShapeDtypeStruct((HIST,), jnp.float32),
               jax.ShapeDtypeStruct((HIST,), jnp.float32)),
)


def _lay1_body(u2s_r, t0_h, t1_h, dis1_h, b1t, z128_in, o0, o1,
               sbuf, dbuf, dloc, val0, val1, t0v, t1v,
               a0, a1, dv, ob0, ob1, bv, acc0, acc1):
    c = lax.axis_index("c")
    s = lax.axis_index("s")

    @pl.when(c == 0)
    def _():
        pltpu.sync_copy(z128_in, acc0.at[pl.ds(s * 128, 128)])
        pltpu.sync_copy(z128_in, acc1.at[pl.ds(s * 128, 128)])
        pltpu.sync_copy(t0_h, t0v)
        pltpu.sync_copy(t1_h, t1v)
        plsc.subcore_barrier()
        r0 = s * RPW1
        pltpu.sync_copy(u2s_r.at[0, pl.ds(r0, RPW1), :], sbuf)
        pltpu.sync_copy(u2s_r.at[1, pl.ds(r0, RPW1), :], dbuf)

        def ch(i, carry):
            def gat(j, carry2):
                sv = sbuf[i, pl.ds(j * 16, 16)]
                d = dbuf[i, pl.ds(j * 16, 16)]
                inside = jnp.logical_and(d >= NE, d < N)
                dloc[i, pl.ds(j * 16, 16)] = jnp.where(
                    inside, d - NE, NSUB + (d & 31))
                val0[pl.ds(j * 16, 16)] = plsc.load_gather(t0v, [sv])
                val1[pl.ds(j * 16, 16)] = plsc.load_gather(t1v, [sv])
                return carry2

            lax.fori_loop(0, KB // 16, gat, 0)
            pltpu.sync_copy(val0, acc0.at[dloc.at[i]], add=True)
            pltpu.sync_copy(val1, acc1.at[dloc.at[i]], add=True)
            return carry

        lax.fori_loop(0, RPW1, ch, 0)
        plsc.subcore_barrier()
        pltpu.sync_copy(acc0.at[pl.ds(s * 128, 128)], a0)
        pltpu.sync_copy(acc1.at[pl.ds(s * 128, 128)], a1)
        pltpu.sync_copy(dis1_h.at[pl.ds(NE + s * 128, 128)], dv)
        pltpu.sync_copy(b1t, bv)

        def fin(j, carry):
            dd = dv[pl.ds(j * 16, 16)]
            tt0 = t0v[pl.ds(NE + s * 128 + j * 16, 16)]
            tt1 = t1v[pl.ds(NE + s * 128 + j * 16, 16)]
            u0 = bv[pl.ds(0, 16)] + dd * (tt0 + a0[pl.ds(j * 16, 16)])
            u1 = bv[pl.ds(16, 16)] + dd * (tt1 + a1[pl.ds(j * 16, 16)])
            ob0[pl.ds(j * 16, 16)] = 1.0 / (1.0 + jnp.exp(-u0))
            ob1[pl.ds(j * 16, 16)] = 1.0 / (1.0 + jnp.exp(-u1))
            return carry

        lax.fori_loop(0, 8, fin, 0)
        pltpu.sync_copy(ob0, o0.at[pl.ds(s * 128, 128)])
        pltpu.sync_copy(ob1, o1.at[pl.ds(s * 128, 128)])


_lay1_call = functools.partial(
    pl.kernel,
    out_type=(jax.ShapeDtypeStruct((SUBP,), jnp.float32),
              jax.ShapeDtypeStruct((SUBP,), jnp.float32)),
    mesh=_MESH,
    scratch_types=[
        pltpu.VMEM((RPW1, KB), jnp.int32),
        pltpu.VMEM((RPW1, KB), jnp.int32),
        pltpu.VMEM((RPW1, KB), jnp.int32),
        pltpu.VMEM((KB,), jnp.float32),
        pltpu.VMEM((KB,), jnp.float32),
        pltpu.VMEM((HIST,), jnp.float32),
        pltpu.VMEM((HIST,), jnp.float32),
        pltpu.VMEM((128,), jnp.float32),
        pltpu.VMEM((128,), jnp.float32),
        pltpu.VMEM((128,), jnp.float32),
        pltpu.VMEM((128,), jnp.float32),
        pltpu.VMEM((128,), jnp.float32),
        pltpu.VMEM((2 * 16,), jnp.float32),
        pltpu.VMEM_SHARED((SUBP,), jnp.float32),
        pltpu.VMEM_SHARED((SUBP,), jnp.float32),
    ],
    compiler_params=pltpu.CompilerParams(needs_layout_passes=False),
)(_lay1_body)


def kernel(features, uni_to_sub, sub_to_uni, num_elements, num_subsets,
           W_sub0, b_sub0, W_uni0, b_uni0, W_sub1, b_sub1, W_uni1, b_uni1):
    del num_elements, num_subsets, W_uni1, b_uni1
    npad = EP - E
    pad_src = (jnp.arange(npad, dtype=jnp.int32) * 97) % N
    pad_dst = N + (jnp.arange(npad, dtype=jnp.int32) % (HIST - N))
    pad = jnp.stack([pad_src, pad_dst], axis=0)
    u2s_r = jnp.concatenate([uni_to_sub, pad], axis=1).reshape(2, ROWS, KB)
    s2u_r = jnp.concatenate([sub_to_uni, pad], axis=1).reshape(2, ROWS, KB)
    zeros_z = jnp.zeros((ZCH,), jnp.float32)
    ones_k = jnp.ones((KB,), jnp.float32)
    zrow = jnp.zeros((128, D), jnp.float32)
    z128 = jnp.zeros((128,), jnp.float32)
    b1t = jnp.repeat(b_sub1.reshape(NNET), 16)

    deg1, deg2 = _deg_call(u2s_r, s2u_r, zeros_z, ones_k)
    y1, y2, dis1, dis2 = _prep_call(deg1, deg2, features)
    accS, accU = _scatter_call(y1, y2, u2s_r, s2u_r, zrow)
    t0, t1 = _dense_call(accS, accU, features, dis1, dis2,
                         W_sub0, b_sub0, W_uni0, b_uni0, W_sub1)
    o0, o1 = _lay1_call(u2s_r, t0, t1, dis1, b1t, z128)
    return jnp.stack([o0[:NSUB], o1[:NSUB]], axis=1)


# speedup vs baseline: 78.0451x; 1.3872x over previous
import functools

import jax
import jax.numpy as jnp
from jax import lax
from jax.experimental import pallas as pl
from jax.experimental.pallas import tpu as pltpu
from jax.experimental.pallas import tpu_sc as plsc

N = 10000
NE = 8000
NSUB = 2000
D = 128
E = 320000
NNET = 2

NCORE = 2
NSC = 16
NW = NCORE * NSC

KB = 128
EP = 327680
ROWS = EP // KB
RPW = ROWS // NW
RPW1 = ROWS // NSC

HIST = 10240
ZCH = HIST // NSC
SUBP = 2048
UNIP = 8192

_MESH = plsc.VectorSubcoreMesh(core_axis_name="c", subcore_axis_name="s")


def _deg_body(u2s_r, s2u_r, zeros_in, ones_in, deg1, deg2,
              dbuf, ones_v, hist1, hist2):
    c = lax.axis_index("c")
    s = lax.axis_index("s")
    wid = c * NSC + s
    pltpu.sync_copy(ones_in, ones_v)
    pltpu.sync_copy(zeros_in, hist1.at[pl.ds(s * ZCH, ZCH)])
    pltpu.sync_copy(zeros_in, hist2.at[pl.ds(s * ZCH, ZCH)])
    plsc.subcore_barrier()
    r0 = wid * RPW
    pltpu.sync_copy(u2s_r.at[1, pl.ds(r0, RPW), :], dbuf)

    def ch1(i, carry):
        pltpu.sync_copy(ones_v, hist1.at[dbuf.at[i]], add=True)
        return carry

    lax.fori_loop(0, RPW, ch1, 0)
    pltpu.sync_copy(s2u_r.at[1, pl.ds(r0, RPW), :], dbuf)

    def ch2(i, carry):
        pltpu.sync_copy(ones_v, hist2.at[dbuf.at[i]], add=True)
        return carry

    lax.fori_loop(0, RPW, ch2, 0)
    plsc.subcore_barrier()
    pltpu.sync_copy(hist1.at[pl.ds(s * ZCH, ZCH)],
                    deg1.at[pl.ds(c * HIST + s * ZCH, ZCH)])
    pltpu.sync_copy(hist2.at[pl.ds(s * ZCH, ZCH)],
                    deg2.at[pl.ds(c * HIST + s * ZCH, ZCH)])


_deg_call = functools.partial(
    pl.kernel,
    out_type=(jax.ShapeDtypeStruct((NCORE * HIST,), jnp.float32),
              jax.ShapeDtypeStruct((NCORE * HIST,), jnp.float32)),
    mesh=_MESH,
    scratch_types=[
        pltpu.VMEM((RPW, KB), jnp.int32),
        pltpu.VMEM((KB,), jnp.float32),
        pltpu.VMEM_SHARED((HIST,), jnp.float32),
        pltpu.VMEM_SHARED((HIST,), jnp.float32),
    ],
)(_deg_body)


def _prep_body(deg1_ref, deg2_ref, x_ref, y1_ref, y2_ref, dis1_ref, dis2_ref):
    deg1 = deg1_ref[...]
    deg2 = deg2_ref[...]
    dis1 = lax.rsqrt(deg1[:HIST] + deg1[HIST:] + 1.0)
    dis2 = lax.rsqrt(deg2[:HIST] + deg2[HIST:] + 1.0)
    dis1_ref[...] = dis1
    dis2_ref[...] = dis2
    x = x_ref[...]
    y1_ref[...] = dis1[:N].reshape(N, 1) * x
    y2_ref[...] = dis2[:N].reshape(N, 1) * x


_prep_call = pl.pallas_call(
    _prep_body,
    out_shape=(jax.ShapeDtypeStruct((N, D), jnp.float32),
               jax.ShapeDtypeStruct((N, D), jnp.float32),
               jax.ShapeDtypeStruct((HIST,), jnp.float32),
               jax.ShapeDtypeStruct((HIST,), jnp.float32)),
)


def _make_scatter(accrows, passes):
    per = accrows // NSC
    tail = per % 128

    def body(y, edges_r, zrow_in, acc_out, sbuf, dbuf, dloc, csf, cdf,
             rb0, acc, sg0):
        c = lax.axis_index("c")
        s = lax.axis_index("s")
        wid = c * NSC + s
        r0 = wid * RPW
        lanes = jnp.arange(16, dtype=jnp.int32)
        pltpu.sync_copy(edges_r.at[0, pl.ds(r0, RPW), :], sbuf)
        pltpu.sync_copy(edges_r.at[1, pl.ds(r0, RPW), :], dbuf)

        for h, (in_lo, in_n, dump_base, dump_mask) in enumerate(passes):
            for k in range(per // 128):
                pltpu.sync_copy(zrow_in,
                                acc.at[pl.ds(s * per + k * 128, 128), :])
            if tail:
                pltpu.sync_copy(
                    zrow_in.at[pl.ds(0, tail), :],
                    acc.at[pl.ds(s * per + (per // 128) * 128, tail), :])
            plsc.subcore_barrier()

            def comp(i, ofs):
                def compj(j, ofs2):
                    sv = sbuf[i, pl.ds(j * 16, 16)]
                    d = dbuf[i, pl.ds(j * 16, 16)]
                    m = jnp.logical_and(d >= in_lo, d < in_lo + in_n)
                    plsc.store_compressed(csf.at[pl.ds(ofs2, 16)], sv, mask=m)
                    plsc.store_compressed(cdf.at[pl.ds(ofs2, 16)], d - in_lo,
                                          mask=m)
                    return ofs2 + jnp.max(plsc.all_reduce_population_count(m))

                return lax.fori_loop(0, KB // 16, compj, ofs)

            n = lax.fori_loop(0, RPW, comp, jnp.int32(0))
            full = lanes < 16
            for t in range(KB // 16):
                plsc.store_compressed(csf.at[pl.ds(n + t * 16, 16)],
                                      lanes, mask=full)
                plsc.store_compressed(
                    cdf.at[pl.ds(n + t * 16, 16)],
                    dump_base + ((lanes + t * 16) & dump_mask), mask=full)
            nch = (n + KB - 1) // KB

            def rep(k, carry):
                def repj(j, carry2):
                    dloc[k, pl.ds(j * 16, 16)] = cdf[
                        pl.ds(k * KB + j * 16, 16)]
                    return carry2

                return lax.fori_loop(0, KB // 16, repj, carry)

            lax.fori_loop(0, nch, rep, 0)

            def ch(k, carry):
                cp = pltpu.async_copy(y.at[csf.at[pl.ds(k * KB, KB)]],
                                      rb0, sg0)
                cp.wait()
                pltpu.sync_copy(rb0, acc.at[dloc.at[k]], add=True)
                return carry

            lax.fori_loop(0, nch, ch, 0)
            plsc.subcore_barrier()
            for k in range(per // 128):
                o = s * per + k * 128
                pltpu.sync_copy(acc.at[pl.ds(o, 128), :],
                                acc_out.at[c, h, pl.ds(o, 128), :])
            if tail:
                o = s * per + (per // 128) * 128
                pltpu.sync_copy(acc.at[pl.ds(o, tail), :],
                                acc_out.at[c, h, pl.ds(o, tail), :])

    return functools.partial(
        pl.kernel,
        out_type=jax.ShapeDtypeStruct((NCORE, len(passes), accrows, D),
                                      jnp.float32),
        mesh=_MESH,
        scratch_types=[
            pltpu.VMEM((RPW, KB), jnp.int32),
            pltpu.VMEM((RPW, KB), jnp.int32),
            pltpu.VMEM((RPW, KB), jnp.int32),
            pltpu.VMEM((RPW * KB + KB,), jnp.int32),
            pltpu.VMEM((RPW * KB + KB,), jnp.int32),
            pltpu.VMEM((KB, D), jnp.float32),
            pltpu.VMEM_SHARED((accrows, D), jnp.float32),
            pltpu.SemaphoreType.DMA,
        ],
        compiler_params=pltpu.CompilerParams(needs_layout_passes=False),
    )(body)


UNIH = NE // 2
UNIPC = 4224
_scatter_sub_call = _make_scatter(SUBP, [(NE, NSUB, NSUB, 31)])
_scatter_uni_call = _make_scatter(
    UNIPC, [(0, UNIH, UNIH, 127), (UNIH, UNIH, UNIH, 127)])


def _dense_body(accS_ref, accU_ref, x_ref, dis1_ref, dis2_ref,
                Ws0_ref, bs0_ref, Wu0_ref, bu0_ref, Ws1_ref,
                t0_ref, t1_ref):
    accS = accS_ref[...]
    accU = accU_ref[...]
    x = x_ref[...]
    dis1 = dis1_ref[...]
    dis2 = dis2_ref[...]
    sumS = accS[0, 0, :NSUB] + accS[1, 0, :NSUB]
    sumU = jnp.concatenate(
        [accU[0, 0, :UNIH] + accU[1, 0, :UNIH],
         accU[0, 1, :UNIH] + accU[1, 1, :UNIH]], axis=0)
    dS = dis1[NE:N].reshape(NSUB, 1)
    aggS = dS * sumS + (dS * dS) * x[NE:]
    dU = dis2[:NE].reshape(NE, 1)
    aggU = dU * sumU + (dU * dU) * x[:NE]
    d1 = dis1[:N]
    t_refs = (t0_ref, t1_ref)
    for i in range(NNET):
        S = jnp.dot(aggS, Ws0_ref[i].T, preferred_element_type=jnp.float32) + bs0_ref[i]
        U = jnp.dot(aggU, Wu0_ref[i].T, preferred_element_type=jnp.float32) + bu0_ref[i]
        f = jnp.concatenate([U, S], axis=0)
        f = 1.0 / (1.0 + jnp.exp(-f))
        z = jnp.dot(f, Ws1_ref[i].reshape(D, 1), preferred_element_type=jnp.float32)
        t = z[:, 0] * d1
        t_refs[i][...] = jnp.concatenate([t, jnp.zeros((HIST - N,), jnp.float32)])


_dense_call = pl.pallas_call(
    _dense_body,
    out_shape=(jax.ShapeDtypeStruct((HIST,), jnp.float32),
               jax.ShapeDtypeStruct((HIST,), jnp.float32)),
)


def _lay1_body(u2s_r, t0_h, t1_h, dis1_h, b1t, z128_in, o0, o1,
               sbuf, dbuf, dloc, csf, cdf, val0, val1, t0v, t1v,
               a0, a1, dv, ob0, ob1, bv, acc0, acc1):
    c = lax.axis_index("c")
    s = lax.axis_index("s")

    @pl.when(c == 0)
    def _():
        pltpu.sync_copy(z128_in, acc0.at[pl.ds(s * 128, 128)])
        pltpu.sync_copy(z128_in, acc1.at[pl.ds(s * 128, 128)])
        pltpu.sync_copy(t0_h, t0v)
        pltpu.sync_copy(t1_h, t1v)
        plsc.subcore_barrier()
        lanes = jnp.arange(16, dtype=jnp.int32)

        def comp_half(h, ofs_init):
            r0 = s * RPW1 + h * (RPW1 // 2)
            pltpu.sync_copy(u2s_r.at[0, pl.ds(r0, RPW1 // 2), :], sbuf)
            pltpu.sync_copy(u2s_r.at[1, pl.ds(r0, RPW1 // 2), :], dbuf)

            def comp(i, ofs):
                def compj(j, ofs2):
                    sv = sbuf[i, pl.ds(j * 16, 16)]
                    d = dbuf[i, pl.ds(j * 16, 16)]
                    m = jnp.logical_and(d >= NE, d < N)
                    plsc.store_compressed(csf.at[pl.ds(ofs2, 16)], sv, mask=m)
                    plsc.store_compressed(cdf.at[pl.ds(ofs2, 16)], d - NE,
                                          mask=m)
                    return ofs2 + jnp.max(plsc.all_reduce_population_count(m))

                return lax.fori_loop(0, KB // 16, compj, ofs)

            return lax.fori_loop(0, RPW1 // 2, comp, ofs_init)

        n = comp_half(0, jnp.int32(0))
        n = comp_half(1, n)
        full = lanes < 16
        for t in range(KB // 16):
            plsc.store_compressed(csf.at[pl.ds(n + t * 16, 16)],
                                  lanes, mask=full)
            plsc.store_compressed(cdf.at[pl.ds(n + t * 16, 16)],
                                  NSUB + ((lanes + t * 16) & 31), mask=full)
        nch = (n + KB - 1) // KB

        def rep(k, carry):
            def repj(j, carry2):
                dloc[k, pl.ds(j * 16, 16)] = cdf[pl.ds(k * KB + j * 16, 16)]
                return carry2

            return lax.fori_loop(0, KB // 16, repj, carry)

        lax.fori_loop(0, nch, rep, 0)

        def ch(k, carry):
            def gat(j, carry2):
                sv = csf[pl.ds(k * KB + j * 16, 16)]
                val0[pl.ds(j * 16, 16)] = plsc.load_gather(t0v, [sv])
                val1[pl.ds(j * 16, 16)] = plsc.load_gather(t1v, [sv])
                return carry2

            lax.fori_loop(0, KB // 16, gat, 0)
            pltpu.sync_copy(val0, acc0.at[dloc.at[k]], add=True)
            pltpu.sync_copy(val1, acc1.at[dloc.at[k]], add=True)
            return carry

        lax.fori_loop(0, nch, ch, 0)
        plsc.subcore_barrier()
        pltpu.sync_copy(acc0.at[pl.ds(s * 128, 128)], a0)
        pltpu.sync_copy(acc1.at[pl.ds(s * 128, 128)], a1)
        pltpu.sync_copy(dis1_h.at[pl.ds(NE + s * 128, 128)], dv)
        pltpu.sync_copy(b1t, bv)

        def fin(j, carry):
            dd = dv[pl.ds(j * 16, 16)]
            tt0 = t0v[pl.ds(NE + s * 128 + j * 16, 16)]
            tt1 = t1v[pl.ds(NE + s * 128 + j * 16, 16)]
            u0 = bv[pl.ds(0, 16)] + dd * (tt0 + a0[pl.ds(j * 16, 16)])
            u1 = bv[pl.ds(16, 16)] + dd * (tt1 + a1[pl.ds(j * 16, 16)])
            ob0[pl.ds(j * 16, 16)] = 1.0 / (1.0 + jnp.exp(-u0))
            ob1[pl.ds(j * 16, 16)] = 1.0 / (1.0 + jnp.exp(-u1))
            return carry

        lax.fori_loop(0, 8, fin, 0)
        pltpu.sync_copy(ob0, o0.at[pl.ds(s * 128, 128)])
        pltpu.sync_copy(ob1, o1.at[pl.ds(s * 128, 128)])


_lay1_call = functools.partial(
    pl.kernel,
    out_type=(jax.ShapeDtypeStruct((SUBP,), jnp.float32),
              jax.ShapeDtypeStruct((SUBP,), jnp.float32)),
    mesh=_MESH,
    scratch_types=[
        pltpu.VMEM((RPW1 // 2, KB), jnp.int32),
        pltpu.VMEM((RPW1 // 2, KB), jnp.int32),
        pltpu.VMEM((RPW1, KB), jnp.int32),
        pltpu.VMEM((RPW1 * KB + KB,), jnp.int32),
        pltpu.VMEM((RPW1 * KB + KB,), jnp.int32),
        pltpu.VMEM((KB,), jnp.float32),
        pltpu.VMEM((KB,), jnp.float32),
        pltpu.VMEM((HIST,), jnp.float32),
        pltpu.VMEM((HIST,), jnp.float32),
        pltpu.VMEM((128,), jnp.float32),
        pltpu.VMEM((128,), jnp.float32),
        pltpu.VMEM((128,), jnp.float32),
        pltpu.VMEM((128,), jnp.float32),
        pltpu.VMEM((128,), jnp.float32),
        pltpu.VMEM((2 * 16,), jnp.float32),
        pltpu.VMEM_SHARED((SUBP,), jnp.float32),
        pltpu.VMEM_SHARED((SUBP,), jnp.float32),
    ],
    compiler_params=pltpu.CompilerParams(needs_layout_passes=False),
)(_lay1_body)


def kernel(features, uni_to_sub, sub_to_uni, num_elements, num_subsets,
           W_sub0, b_sub0, W_uni0, b_uni0, W_sub1, b_sub1, W_uni1, b_uni1):
    del num_elements, num_subsets, W_uni1, b_uni1
    npad = EP - E
    pad_src = (jnp.arange(npad, dtype=jnp.int32) * 97) % N
    pad_dst = N + (jnp.arange(npad, dtype=jnp.int32) % (HIST - N))
    pad = jnp.stack([pad_src, pad_dst], axis=0)
    u2s_r = jnp.concatenate([uni_to_sub, pad], axis=1).reshape(2, ROWS, KB)
    s2u_r = jnp.concatenate([sub_to_uni, pad], axis=1).reshape(2, ROWS, KB)
    zeros_z = jnp.zeros((ZCH,), jnp.float32)
    ones_k = jnp.ones((KB,), jnp.float32)
    zrow = jnp.zeros((128, D), jnp.float32)
    z128 = jnp.zeros((128,), jnp.float32)
    b1t = jnp.repeat(b_sub1.reshape(NNET), 16)

    deg1, deg2 = _deg_call(u2s_r, s2u_r, zeros_z, ones_k)
    y1, y2, dis1, dis2 = _prep_call(deg1, deg2, features)
    accS = _scatter_sub_call(y1, u2s_r, zrow)
    accU = _scatter_uni_call(y2, s2u_r, zrow)
    t0, t1 = _dense_call(accS, accU, features, dis1, dis2,
                         W_sub0, b_sub0, W_uni0, b_uni0, W_sub1)
    o0, o1 = _lay1_call(u2s_r, t0, t1, dis1, b1t, z128)
    return jnp.stack([o0[:NSUB], o1[:NSUB]], axis=1)


# speedup vs baseline: 89.3193x; 1.1445x over previous
import functools

import jax
import jax.numpy as jnp
from jax import lax
from jax.experimental import pallas as pl
from jax.experimental.pallas import tpu as pltpu
from jax.experimental.pallas import tpu_sc as plsc

N = 10000
NE = 8000
NSUB = 2000
D = 128
E = 320000
NNET = 2

NCORE = 2
NSC = 16
NW = NCORE * NSC

KB = 128
EP = 327680
ROWS = EP // KB
RPW = ROWS // NW
RPW1 = ROWS // NSC

HIST = 10240
ZCH = HIST // NSC
SUBP = 2048
UNIP = 8192

_MESH = plsc.VectorSubcoreMesh(core_axis_name="c", subcore_axis_name="s")


def _deg_body(u2s_r, s2u_r, zeros_in, ones_in, deg1, deg2,
              dbuf, ones_v, hist1, hist2):
    c = lax.axis_index("c")
    s = lax.axis_index("s")
    wid = c * NSC + s
    pltpu.sync_copy(ones_in, ones_v)
    pltpu.sync_copy(zeros_in, hist1.at[pl.ds(s * ZCH, ZCH)])
    pltpu.sync_copy(zeros_in, hist2.at[pl.ds(s * ZCH, ZCH)])
    plsc.subcore_barrier()
    r0 = wid * RPW
    pltpu.sync_copy(u2s_r.at[1, pl.ds(r0, RPW), :], dbuf)

    def ch1(i, carry):
        pltpu.sync_copy(ones_v, hist1.at[dbuf.at[i]], add=True)
        return carry

    lax.fori_loop(0, RPW, ch1, 0)
    pltpu.sync_copy(s2u_r.at[1, pl.ds(r0, RPW), :], dbuf)

    def ch2(i, carry):
        pltpu.sync_copy(ones_v, hist2.at[dbuf.at[i]], add=True)
        return carry

    lax.fori_loop(0, RPW, ch2, 0)
    plsc.subcore_barrier()
    pltpu.sync_copy(hist1.at[pl.ds(s * ZCH, ZCH)],
                    deg1.at[pl.ds(c * HIST + s * ZCH, ZCH)])
    pltpu.sync_copy(hist2.at[pl.ds(s * ZCH, ZCH)],
                    deg2.at[pl.ds(c * HIST + s * ZCH, ZCH)])


_deg_call = functools.partial(
    pl.kernel,
    out_type=(jax.ShapeDtypeStruct((NCORE * HIST,), jnp.float32),
              jax.ShapeDtypeStruct((NCORE * HIST,), jnp.float32)),
    mesh=_MESH,
    scratch_types=[
        pltpu.VMEM((RPW, KB), jnp.int32),
        pltpu.VMEM((KB,), jnp.float32),
        pltpu.VMEM_SHARED((HIST,), jnp.float32),
        pltpu.VMEM_SHARED((HIST,), jnp.float32),
    ],
)(_deg_body)


def _prep_body(deg1_ref, deg2_ref, x_ref, y1_ref, y2_ref, dis1_ref, dis2_ref):
    deg1 = deg1_ref[...]
    deg2 = deg2_ref[...]
    dis1 = lax.rsqrt(deg1[:HIST] + deg1[HIST:] + 1.0)
    dis2 = lax.rsqrt(deg2[:HIST] + deg2[HIST:] + 1.0)
    dis1_ref[...] = dis1
    dis2_ref[...] = dis2
    x = x_ref[...]
    y1_ref[...] = dis1[:N].reshape(N, 1) * x
    y2_ref[...] = dis2[:N].reshape(N, 1) * x


_prep_call = pl.pallas_call(
    _prep_body,
    out_shape=(jax.ShapeDtypeStruct((N, D), jnp.float32),
               jax.ShapeDtypeStruct((N, D), jnp.float32),
               jax.ShapeDtypeStruct((HIST,), jnp.float32),
               jax.ShapeDtypeStruct((HIST,), jnp.float32)),
)


def _make_scatter(accrows, passes):
    per = accrows // NSC
    tail = per % 128

    def body(y, edges_r, zrow_in, acc_out, sbuf, dbuf, dloc, csf, cdf,
             rb0, rb1, acc, sg0, sg1):
        c = lax.axis_index("c")
        s = lax.axis_index("s")
        wid = c * NSC + s
        r0 = wid * RPW
        lanes = jnp.arange(16, dtype=jnp.int32)
        pltpu.sync_copy(edges_r.at[0, pl.ds(r0, RPW), :], sbuf)
        pltpu.sync_copy(edges_r.at[1, pl.ds(r0, RPW), :], dbuf)

        for h, (in_lo, in_n, dump_base, dump_mask) in enumerate(passes):
            for k in range(per // 128):
                pltpu.sync_copy(zrow_in,
                                acc.at[pl.ds(s * per + k * 128, 128), :])
            if tail:
                pltpu.sync_copy(
                    zrow_in.at[pl.ds(0, tail), :],
                    acc.at[pl.ds(s * per + (per // 128) * 128, tail), :])
            plsc.subcore_barrier()

            def comp(i, ofs):
                def compj(j, ofs2):
                    sv = sbuf[i, pl.ds(j * 16, 16)]
                    d = dbuf[i, pl.ds(j * 16, 16)]
                    m = jnp.logical_and(d >= in_lo, d < in_lo + in_n)
                    plsc.store_compressed(csf.at[pl.ds(ofs2, 16)], sv, mask=m)
                    plsc.store_compressed(cdf.at[pl.ds(ofs2, 16)], d - in_lo,
                                          mask=m)
                    return ofs2 + jnp.max(plsc.all_reduce_population_count(m))

                return lax.fori_loop(0, KB // 16, compj, ofs)

            n = lax.fori_loop(0, RPW, comp, jnp.int32(0))
            full = lanes < 16
            for t in range(KB // 16):
                plsc.store_compressed(csf.at[pl.ds(n + t * 16, 16)],
                                      lanes, mask=full)
                plsc.store_compressed(
                    cdf.at[pl.ds(n + t * 16, 16)],
                    dump_base + ((lanes + t * 16) & dump_mask), mask=full)
            nch = (n + KB - 1) // KB

            def rep(k, carry):
                def repj(j, carry2):
                    dloc[k, pl.ds(j * 16, 16)] = cdf[
                        pl.ds(k * KB + j * 16, 16)]
                    return carry2

                return lax.fori_loop(0, KB // 16, repj, carry)

            lax.fori_loop(0, nch, rep, 0)

            @pl.when(nch > 0)
            def _():
                pltpu.async_copy(y.at[csf.at[pl.ds(0, KB)]], rb0, sg0)

            def ch(kk, carry):
                k0 = 2 * kk
                k1 = 2 * kk + 1
                pltpu.make_async_copy(y.at[csf.at[pl.ds(k0 * KB, KB)]],
                                      rb0, sg0).wait()

                @pl.when(k1 < nch)
                def _():
                    pltpu.async_copy(y.at[csf.at[pl.ds(k1 * KB, KB)]],
                                     rb1, sg1)

                pltpu.sync_copy(rb0, acc.at[dloc.at[k0]], add=True)

                @pl.when(k1 < nch)
                def _():
                    pltpu.make_async_copy(y.at[csf.at[pl.ds(k1 * KB, KB)]],
                                          rb1, sg1).wait()

                    @pl.when(k1 + 1 < nch)
                    def _():
                        pltpu.async_copy(
                            y.at[csf.at[pl.ds((k1 + 1) * KB, KB)]], rb0, sg0)

                    pltpu.sync_copy(rb1, acc.at[dloc.at[k1]], add=True)

                return carry

            lax.fori_loop(0, (nch + 1) // 2, ch, 0)
            plsc.subcore_barrier()
            for k in range(per // 128):
                o = s * per + k * 128
                pltpu.sync_copy(acc.at[pl.ds(o, 128), :],
                                acc_out.at[c, h, pl.ds(o, 128), :])
            if tail:
                o = s * per + (per // 128) * 128
                pltpu.sync_copy(acc.at[pl.ds(o, tail), :],
                                acc_out.at[c, h, pl.ds(o, tail), :])

    return functools.partial(
        pl.kernel,
        out_type=jax.ShapeDtypeStruct((NCORE, len(passes), accrows, D),
                                      jnp.float32),
        mesh=_MESH,
        scratch_types=[
            pltpu.VMEM((RPW, KB), jnp.int32),
            pltpu.VMEM((RPW, KB), jnp.int32),
            pltpu.VMEM((RPW, KB), jnp.int32),
            pltpu.VMEM((RPW * KB + KB,), jnp.int32),
            pltpu.VMEM((RPW * KB + KB,), jnp.int32),
            pltpu.VMEM((KB, D), jnp.float32),
            pltpu.VMEM((KB, D), jnp.float32),
            pltpu.VMEM_SHARED((accrows, D), jnp.float32),
            pltpu.SemaphoreType.DMA,
            pltpu.SemaphoreType.DMA,
        ],
        compiler_params=pltpu.CompilerParams(needs_layout_passes=False),
    )(body)


UNIH = NE // 2
UNIPC = 4224
_scatter_sub_call = _make_scatter(SUBP, [(NE, NSUB, NSUB, 31)])
_scatter_uni_call = _make_scatter(
    UNIPC, [(0, UNIH, UNIH, 127), (UNIH, UNIH, UNIH, 127)])


def _dense_body(accS_ref, accU_ref, x_ref, dis1_ref, dis2_ref,
                Ws0_ref, bs0_ref, Wu0_ref, bu0_ref, Ws1_ref,
                t0_ref, t1_ref):
    accS = accS_ref[...]
    accU = accU_ref[...]
    x = x_ref[...]
    dis1 = dis1_ref[...]
    dis2 = dis2_ref[...]
    sumS = accS[0, 0, :NSUB] + accS[1, 0, :NSUB]
    sumU = jnp.concatenate(
        [accU[0, 0, :UNIH] + accU[1, 0, :UNIH],
         accU[0, 1, :UNIH] + accU[1, 1, :UNIH]], axis=0)
    dS = dis1[NE:N].reshape(NSUB, 1)
    aggS = dS * sumS + (dS * dS) * x[NE:]
    dU = dis2[:NE].reshape(NE, 1)
    aggU = dU * sumU + (dU * dU) * x[:NE]
    d1 = dis1[:N]
    t_refs = (t0_ref, t1_ref)
    for i in range(NNET):
        S = jnp.dot(aggS, Ws0_ref[i].T, preferred_element_type=jnp.float32) + bs0_ref[i]
        U = jnp.dot(aggU, Wu0_ref[i].T, preferred_element_type=jnp.float32) + bu0_ref[i]
        f = jnp.concatenate([U, S], axis=0)
        f = 1.0 / (1.0 + jnp.exp(-f))
        z = jnp.dot(f, Ws1_ref[i].reshape(D, 1), preferred_element_type=jnp.float32)
        t = z[:, 0] * d1
        t_refs[i][...] = jnp.concatenate([t, jnp.zeros((HIST - N,), jnp.float32)])


_dense_call = pl.pallas_call(
    _dense_body,
    out_shape=(jax.ShapeDtypeStruct((HIST,), jnp.float32),
               jax.ShapeDtypeStruct((HIST,), jnp.float32)),
)


def _lay1_body(u2s_r, t0_h, t1_h, dis1_h, b1t, z128_in, o0, o1,
               sbuf, dbuf, dloc, csf, cdf, val0, val1, t0v, t1v,
               a0, a1, dv, ob0, ob1, bv, acc0, acc1):
    c = lax.axis_index("c")
    s = lax.axis_index("s")

    @pl.when(c == 0)
    def _():
        pltpu.sync_copy(z128_in, acc0.at[pl.ds(s * 128, 128)])
        pltpu.sync_copy(z128_in, acc1.at[pl.ds(s * 128, 128)])
        pltpu.sync_copy(t0_h, t0v)
        pltpu.sync_copy(t1_h, t1v)
        plsc.subcore_barrier()
        lanes = jnp.arange(16, dtype=jnp.int32)

        def comp_half(h, ofs_init):
            r0 = s * RPW1 + h * (RPW1 // 2)
            pltpu.sync_copy(u2s_r.at[0, pl.ds(r0, RPW1 // 2), :], sbuf)
            pltpu.sync_copy(u2s_r.at[1, pl.ds(r0, RPW1 // 2), :], dbuf)

            def comp(i, ofs):
                def compj(j, ofs2):
                    sv = sbuf[i, pl.ds(j * 16, 16)]
                    d = dbuf[i, pl.ds(j * 16, 16)]
                    m = jnp.logical_and(d >= NE, d < N)
                    plsc.store_compressed(csf.at[pl.ds(ofs2, 16)], sv, mask=m)
                    plsc.store_compressed(cdf.at[pl.ds(ofs2, 16)], d - NE,
                                          mask=m)
                    return ofs2 + jnp.max(plsc.all_reduce_population_count(m))

                return lax.fori_loop(0, KB // 16, compj, ofs)

            return lax.fori_loop(0, RPW1 // 2, comp, ofs_init)

        n = comp_half(0, jnp.int32(0))
        n = comp_half(1, n)
        full = lanes < 16
        for t in range(KB // 16):
            plsc.store_compressed(csf.at[pl.ds(n + t * 16, 16)],
                                  lanes, mask=full)
            plsc.store_compressed(cdf.at[pl.ds(n + t * 16, 16)],
                                  NSUB + ((lanes + t * 16) & 31), mask=full)
        nch = (n + KB - 1) // KB

        def rep(k, carry):
            def repj(j, carry2):
                dloc[k, pl.ds(j * 16, 16)] = cdf[pl.ds(k * KB + j * 16, 16)]
                return carry2

            return lax.fori_loop(0, KB // 16, repj, carry)

        lax.fori_loop(0, nch, rep, 0)

        def ch(k, carry):
            def gat(j, carry2):
                sv = csf[pl.ds(k * KB + j * 16, 16)]
                val0[pl.ds(j * 16, 16)] = plsc.load_gather(t0v, [sv])
                val1[pl.ds(j * 16, 16)] = plsc.load_gather(t1v, [sv])
                return carry2

            lax.fori_loop(0, KB // 16, gat, 0)
            pltpu.sync_copy(val0, acc0.at[dloc.at[k]], add=True)
            pltpu.sync_copy(val1, acc1.at[dloc.at[k]], add=True)
            return carry

        lax.fori_loop(0, nch, ch, 0)
        plsc.subcore_barrier()
        pltpu.sync_copy(acc0.at[pl.ds(s * 128, 128)], a0)
        pltpu.sync_copy(acc1.at[pl.ds(s * 128, 128)], a1)
        pltpu.sync_copy(dis1_h.at[pl.ds(NE + s * 128, 128)], dv)
        pltpu.sync_copy(b1t, bv)

        def fin(j, carry):
            dd = dv[pl.ds(j * 16, 16)]
            tt0 = t0v[pl.ds(NE + s * 128 + j * 16, 16)]
            tt1 = t1v[pl.ds(NE + s * 128 + j * 16, 16)]
            u0 = bv[pl.ds(0, 16)] + dd * (tt0 + a0[pl.ds(j * 16, 16)])
            u1 = bv[pl.ds(16, 16)] + dd * (tt1 + a1[pl.ds(j * 16, 16)])
            ob0[pl.ds(j * 16, 16)] = 1.0 / (1.0 + jnp.exp(-u0))
            ob1[pl.ds(j * 16, 16)] = 1.0 / (1.0 + jnp.exp(-u1))
            return carry

        lax.fori_loop(0, 8, fin, 0)
        pltpu.sync_copy(ob0, o0.at[pl.ds(s * 128, 128)])
        pltpu.sync_copy(ob1, o1.at[pl.ds(s * 128, 128)])


_lay1_call = functools.partial(
    pl.kernel,
    out_type=(jax.ShapeDtypeStruct((SUBP,), jnp.float32),
              jax.ShapeDtypeStruct((SUBP,), jnp.float32)),
    mesh=_MESH,
    scratch_types=[
        pltpu.VMEM((RPW1 // 2, KB), jnp.int32),
        pltpu.VMEM((RPW1 // 2, KB), jnp.int32),
        pltpu.VMEM((RPW1, KB), jnp.int32),
        pltpu.VMEM((RPW1 * KB + KB,), jnp.int32),
        pltpu.VMEM((RPW1 * KB + KB,), jnp.int32),
        pltpu.VMEM((KB,), jnp.float32),
        pltpu.VMEM((KB,), jnp.float32),
        pltpu.VMEM((HIST,), jnp.float32),
        pltpu.VMEM((HIST,), jnp.float32),
        pltpu.VMEM((128,), jnp.float32),
        pltpu.VMEM((128,), jnp.float32),
        pltpu.VMEM((128,), jnp.float32),
        pltpu.VMEM((128,), jnp.float32),
        pltpu.VMEM((128,), jnp.float32),
        pltpu.VMEM((2 * 16,), jnp.float32),
        pltpu.VMEM_SHARED((SUBP,), jnp.float32),
        pltpu.VMEM_SHARED((SUBP,), jnp.float32),
    ],
    compiler_params=pltpu.CompilerParams(needs_layout_passes=False),
)(_lay1_body)


def kernel(features, uni_to_sub, sub_to_uni, num_elements, num_subsets,
           W_sub0, b_sub0, W_uni0, b_uni0, W_sub1, b_sub1, W_uni1, b_uni1):
    del num_elements, num_subsets, W_uni1, b_uni1
    npad = EP - E
    pad_src = (jnp.arange(npad, dtype=jnp.int32) * 97) % N
    pad_dst = N + (jnp.arange(npad, dtype=jnp.int32) % (HIST - N))
    pad = jnp.stack([pad_src, pad_dst], axis=0)
    u2s_r = jnp.concatenate([uni_to_sub, pad], axis=1).reshape(2, ROWS, KB)
    s2u_r = jnp.concatenate([sub_to_uni, pad], axis=1).reshape(2, ROWS, KB)
    zeros_z = jnp.zeros((ZCH,), jnp.float32)
    ones_k = jnp.ones((KB,), jnp.float32)
    zrow = jnp.zeros((128, D), jnp.float32)
    z128 = jnp.zeros((128,), jnp.float32)
    b1t = jnp.repeat(b_sub1.reshape(NNET), 16)

    deg1, deg2 = _deg_call(u2s_r, s2u_r, zeros_z, ones_k)
    y1, y2, dis1, dis2 = _prep_call(deg1, deg2, features)
    accS = _scatter_sub_call(y1, u2s_r, zrow)
    accU = _scatter_uni_call(y2, s2u_r, zrow)
    t0, t1 = _dense_call(accS, accU, features, dis1, dis2,
                         W_sub0, b_sub0, W_uni0, b_uni0, W_sub1)
    o0, o1 = _lay1_call(u2s_r, t0, t1, dis1, b1t, z128)
    return jnp.stack([o0[:NSUB], o1[:NSUB]], axis=1)


# speedup vs baseline: 92.4902x; 1.0355x over previous
import functools

import jax
import jax.numpy as jnp
from jax import lax
from jax.experimental import pallas as pl
from jax.experimental.pallas import tpu as pltpu
from jax.experimental.pallas import tpu_sc as plsc

N = 10000
NE = 8000
NSUB = 2000
D = 128
E = 320000
NNET = 2

NCORE = 2
NSC = 16
NW = NCORE * NSC

KB = 128
EP = 327680
ROWS = EP // KB
RPW = ROWS // NW
RPW1 = ROWS // NSC

HIST = 10240
ZCH = HIST // NSC
SUBP = 2048
UNIP = 8192

_MESH = plsc.VectorSubcoreMesh(core_axis_name="c", subcore_axis_name="s")


def _deg_body(u2s_r, s2u_r, zeros_in, ones_in, deg1, deg2,
              dbuf, ones_v, hist1, hist2, sg):
    c = lax.axis_index("c")
    s = lax.axis_index("s")
    wid = c * NSC + s
    pltpu.sync_copy(ones_in, ones_v)
    pltpu.sync_copy(zeros_in, hist1.at[pl.ds(s * ZCH, ZCH)])
    pltpu.sync_copy(zeros_in, hist2.at[pl.ds(s * ZCH, ZCH)])
    plsc.subcore_barrier()
    r0 = wid * RPW
    pltpu.sync_copy(u2s_r.at[1, pl.ds(r0, RPW), :], dbuf)

    def ch1(i, carry):
        pltpu.async_copy(ones_v, hist1.at[dbuf.at[i]], sg, add=True)
        return carry

    lax.fori_loop(0, RPW, ch1, 0)
    pltpu.make_async_copy(u2s_r.at[1, pl.ds(r0, RPW), :], dbuf, sg).wait()
    pltpu.sync_copy(s2u_r.at[1, pl.ds(r0, RPW), :], dbuf)

    def ch2(i, carry):
        pltpu.async_copy(ones_v, hist2.at[dbuf.at[i]], sg, add=True)
        return carry

    lax.fori_loop(0, RPW, ch2, 0)
    pltpu.make_async_copy(s2u_r.at[1, pl.ds(r0, RPW), :], dbuf, sg).wait()
    plsc.subcore_barrier()
    pltpu.sync_copy(hist1.at[pl.ds(s * ZCH, ZCH)],
                    deg1.at[pl.ds(c * HIST + s * ZCH, ZCH)])
    pltpu.sync_copy(hist2.at[pl.ds(s * ZCH, ZCH)],
                    deg2.at[pl.ds(c * HIST + s * ZCH, ZCH)])


_deg_call = functools.partial(
    pl.kernel,
    out_type=(jax.ShapeDtypeStruct((NCORE * HIST,), jnp.float32),
              jax.ShapeDtypeStruct((NCORE * HIST,), jnp.float32)),
    mesh=_MESH,
    scratch_types=[
        pltpu.VMEM((RPW, KB), jnp.int32),
        pltpu.VMEM((KB,), jnp.float32),
        pltpu.VMEM_SHARED((HIST,), jnp.float32),
        pltpu.VMEM_SHARED((HIST,), jnp.float32),
        pltpu.SemaphoreType.DMA,
    ],
)(_deg_body)


def _prep_body(deg1_ref, deg2_ref, x_ref, y1_ref, y2_ref, dis1_ref, dis2_ref):
    deg1 = deg1_ref[...]
    deg2 = deg2_ref[...]
    dis1 = lax.rsqrt(deg1[:HIST] + deg1[HIST:] + 1.0)
    dis2 = lax.rsqrt(deg2[:HIST] + deg2[HIST:] + 1.0)
    dis1_ref[...] = dis1
    dis2_ref[...] = dis2
    x = x_ref[...]
    y1_ref[...] = dis1[:N].reshape(N, 1) * x
    y2_ref[...] = dis2[:N].reshape(N, 1) * x


_prep_call = pl.pallas_call(
    _prep_body,
    out_shape=(jax.ShapeDtypeStruct((N, D), jnp.float32),
               jax.ShapeDtypeStruct((N, D), jnp.float32),
               jax.ShapeDtypeStruct((HIST,), jnp.float32),
               jax.ShapeDtypeStruct((HIST,), jnp.float32)),
)


def _make_scatter(accrows, passes):
    per = accrows // NSC
    tail = per % 128

    def body(y, edges_r, zrow_in, acc_out, sbuf, dbuf, dloc, csf, cdf,
             rb0, rb1, acc, sg0, sg1):
        c = lax.axis_index("c")
        s = lax.axis_index("s")
        wid = c * NSC + s
        r0 = wid * RPW
        lanes = jnp.arange(16, dtype=jnp.int32)
        pltpu.sync_copy(edges_r.at[0, pl.ds(r0, RPW), :], sbuf)
        pltpu.sync_copy(edges_r.at[1, pl.ds(r0, RPW), :], dbuf)

        for h, (in_lo, in_n, dump_base, dump_mask) in enumerate(passes):
            for k in range(per // 128):
                pltpu.sync_copy(zrow_in,
                                acc.at[pl.ds(s * per + k * 128, 128), :])
            if tail:
                pltpu.sync_copy(
                    zrow_in.at[pl.ds(0, tail), :],
                    acc.at[pl.ds(s * per + (per // 128) * 128, tail), :])
            plsc.subcore_barrier()

            def comp(i, ofs):
                def compj(j, ofs2):
                    sv = sbuf[i, pl.ds(j * 16, 16)]
                    d = dbuf[i, pl.ds(j * 16, 16)]
                    m = jnp.logical_and(d >= in_lo, d < in_lo + in_n)
                    plsc.store_compressed(csf.at[pl.ds(ofs2, 16)], sv, mask=m)
                    plsc.store_compressed(cdf.at[pl.ds(ofs2, 16)], d - in_lo,
                                          mask=m)
                    return ofs2 + jnp.max(plsc.all_reduce_population_count(m))

                return lax.fori_loop(0, KB // 16, compj, ofs)

            n = lax.fori_loop(0, RPW, comp, jnp.int32(0))
            full = lanes < 16
            for t in range(KB // 16):
                plsc.store_compressed(csf.at[pl.ds(n + t * 16, 16)],
                                      lanes, mask=full)
                plsc.store_compressed(
                    cdf.at[pl.ds(n + t * 16, 16)],
                    dump_base + ((lanes + t * 16) & dump_mask), mask=full)
            nch = (n + KB - 1) // KB

            def rep(k, carry):
                def repj(j, carry2):
                    dloc[k, pl.ds(j * 16, 16)] = cdf[
                        pl.ds(k * KB + j * 16, 16)]
                    return carry2

                return lax.fori_loop(0, KB // 16, repj, carry)

            lax.fori_loop(0, nch, rep, 0)

            @pl.when(nch > 0)
            def _():
                pltpu.async_copy(y.at[csf.at[pl.ds(0, KB)]], rb0, sg0)

            def ch(kk, carry):
                k0 = 2 * kk
                k1 = 2 * kk + 1
                pltpu.make_async_copy(y.at[csf.at[pl.ds(k0 * KB, KB)]],
                                      rb0, sg0).wait()

                @pl.when(k1 < nch)
                def _():
                    pltpu.async_copy(y.at[csf.at[pl.ds(k1 * KB, KB)]],
                                     rb1, sg1)

                pltpu.sync_copy(rb0, acc.at[dloc.at[k0]], add=True)

                @pl.when(k1 < nch)
                def _():
                    pltpu.make_async_copy(y.at[csf.at[pl.ds(k1 * KB, KB)]],
                                          rb1, sg1).wait()

                    @pl.when(k1 + 1 < nch)
                    def _():
                        pltpu.async_copy(
                            y.at[csf.at[pl.ds((k1 + 1) * KB, KB)]], rb0, sg0)

                    pltpu.sync_copy(rb1, acc.at[dloc.at[k1]], add=True)

                return carry

            lax.fori_loop(0, (nch + 1) // 2, ch, 0)
            plsc.subcore_barrier()
            for k in range(per // 128):
                o = s * per + k * 128
                pltpu.sync_copy(acc.at[pl.ds(o, 128), :],
                                acc_out.at[c, h, pl.ds(o, 128), :])
            if tail:
                o = s * per + (per // 128) * 128
                pltpu.sync_copy(acc.at[pl.ds(o, tail), :],
                                acc_out.at[c, h, pl.ds(o, tail), :])

    return functools.partial(
        pl.kernel,
        out_type=jax.ShapeDtypeStruct((NCORE, len(passes), accrows, D),
                                      jnp.float32),
        mesh=_MESH,
        scratch_types=[
            pltpu.VMEM((RPW, KB), jnp.int32),
            pltpu.VMEM((RPW, KB), jnp.int32),
            pltpu.VMEM((RPW, KB), jnp.int32),
            pltpu.VMEM((RPW * KB + KB,), jnp.int32),
            pltpu.VMEM((RPW * KB + KB,), jnp.int32),
            pltpu.VMEM((KB, D), jnp.float32),
            pltpu.VMEM((KB, D), jnp.float32),
            pltpu.VMEM_SHARED((accrows, D), jnp.float32),
            pltpu.SemaphoreType.DMA,
            pltpu.SemaphoreType.DMA,
        ],
        compiler_params=pltpu.CompilerParams(needs_layout_passes=False),
    )(body)


UNIH = NE // 2
UNIPC = 4224
_scatter_sub_call = _make_scatter(SUBP, [(NE, NSUB, NSUB, 31)])
_scatter_uni_call = _make_scatter(
    UNIPC, [(0, UNIH, UNIH, 127), (UNIH, UNIH, UNIH, 127)])


def _dense_body(accS_ref, accU_ref, x_ref, dis1_ref, dis2_ref,
                Ws0_ref, bs0_ref, Wu0_ref, bu0_ref, Ws1_ref,
                t0_ref, t1_ref):
    accS = accS_ref[...]
    accU = accU_ref[...]
    x = x_ref[...]
    dis1 = dis1_ref[...]
    dis2 = dis2_ref[...]
    sumS = accS[0, 0, :NSUB] + accS[1, 0, :NSUB]
    sumU = jnp.concatenate(
        [accU[0, 0, :UNIH] + accU[1, 0, :UNIH],
         accU[0, 1, :UNIH] + accU[1, 1, :UNIH]], axis=0)
    dS = dis1[NE:N].reshape(NSUB, 1)
    aggS = dS * sumS + (dS * dS) * x[NE:]
    dU = dis2[:NE].reshape(NE, 1)
    aggU = dU * sumU + (dU * dU) * x[:NE]
    d1 = dis1[:N]
    t_refs = (t0_ref, t1_ref)
    for i in range(NNET):
        S = jnp.dot(aggS, Ws0_ref[i].T, preferred_element_type=jnp.float32) + bs0_ref[i]
        U = jnp.dot(aggU, Wu0_ref[i].T, preferred_element_type=jnp.float32) + bu0_ref[i]
        f = jnp.concatenate([U, S], axis=0)
        f = 1.0 / (1.0 + jnp.exp(-f))
        z = jnp.dot(f, Ws1_ref[i].reshape(D, 1), preferred_element_type=jnp.float32)
        t = z[:, 0] * d1
        t_refs[i][...] = jnp.concatenate([t, jnp.zeros((HIST - N,), jnp.float32)])


_dense_call = pl.pallas_call(
    _dense_body,
    out_shape=(jax.ShapeDtypeStruct((HIST,), jnp.float32),
               jax.ShapeDtypeStruct((HIST,), jnp.float32)),
)


def _lay1_body(u2s_r, t0_h, t1_h, dis1_h, b1t, z128_in, o0, o1,
               sbuf, dbuf, dloc, csf, cdf, val0, val1, t0v, t1v,
               a0, a1, dv, ob0, ob1, bv, acc0, acc1):
    c = lax.axis_index("c")
    s = lax.axis_index("s")

    @pl.when(c == 0)
    def _():
        pltpu.sync_copy(z128_in, acc0.at[pl.ds(s * 128, 128)])
        pltpu.sync_copy(z128_in, acc1.at[pl.ds(s * 128, 128)])
        pltpu.sync_copy(t0_h, t0v)
        pltpu.sync_copy(t1_h, t1v)
        plsc.subcore_barrier()
        lanes = jnp.arange(16, dtype=jnp.int32)

        def comp_half(h, ofs_init):
            r0 = s * RPW1 + h * (RPW1 // 2)
            pltpu.sync_copy(u2s_r.at[0, pl.ds(r0, RPW1 // 2), :], sbuf)
            pltpu.sync_copy(u2s_r.at[1, pl.ds(r0, RPW1 // 2), :], dbuf)

            def comp(i, ofs):
                def compj(j, ofs2):
                    sv = sbuf[i, pl.ds(j * 16, 16)]
                    d = dbuf[i, pl.ds(j * 16, 16)]
                    m = jnp.logical_and(d >= NE, d < N)
                    plsc.store_compressed(csf.at[pl.ds(ofs2, 16)], sv, mask=m)
                    plsc.store_compressed(cdf.at[pl.ds(ofs2, 16)], d - NE,
                                          mask=m)
                    return ofs2 + jnp.max(plsc.all_reduce_population_count(m))

                return lax.fori_loop(0, KB // 16, compj, ofs)

            return lax.fori_loop(0, RPW1 // 2, comp, ofs_init)

        n = comp_half(0, jnp.int32(0))
        n = comp_half(1, n)
        full = lanes < 16
        for t in range(KB // 16):
            plsc.store_compressed(csf.at[pl.ds(n + t * 16, 16)],
                                  lanes, mask=full)
            plsc.store_compressed(cdf.at[pl.ds(n + t * 16, 16)],
                                  NSUB + ((lanes + t * 16) & 31), mask=full)
        nch = (n + KB - 1) // KB

        def rep(k, carry):
            def repj(j, carry2):
                dloc[k, pl.ds(j * 16, 16)] = cdf[pl.ds(k * KB + j * 16, 16)]
                return carry2

            return lax.fori_loop(0, KB // 16, repj, carry)

        lax.fori_loop(0, nch, rep, 0)

        def ch(k, carry):
            def gat(j, carry2):
                sv = csf[pl.ds(k * KB + j * 16, 16)]
                val0[pl.ds(j * 16, 16)] = plsc.load_gather(t0v, [sv])
                val1[pl.ds(j * 16, 16)] = plsc.load_gather(t1v, [sv])
                return carry2

            lax.fori_loop(0, KB // 16, gat, 0)
            pltpu.sync_copy(val0, acc0.at[dloc.at[k]], add=True)
            pltpu.sync_copy(val1, acc1.at[dloc.at[k]], add=True)
            return carry

        lax.fori_loop(0, nch, ch, 0)
        plsc.subcore_barrier()
        pltpu.sync_copy(acc0.at[pl.ds(s * 128, 128)], a0)
        pltpu.sync_copy(acc1.at[pl.ds(s * 128, 128)], a1)
        pltpu.sync_copy(dis1_h.at[pl.ds(NE + s * 128, 128)], dv)
        pltpu.sync_copy(b1t, bv)

        def fin(j, carry):
            dd = dv[pl.ds(j * 16, 16)]
            tt0 = t0v[pl.ds(NE + s * 128 + j * 16, 16)]
            tt1 = t1v[pl.ds(NE + s * 128 + j * 16, 16)]
            u0 = bv[pl.ds(0, 16)] + dd * (tt0 + a0[pl.ds(j * 16, 16)])
            u1 = bv[pl.ds(16, 16)] + dd * (tt1 + a1[pl.ds(j * 16, 16)])
            ob0[pl.ds(j * 16, 16)] = 1.0 / (1.0 + jnp.exp(-u0))
            ob1[pl.ds(j * 16, 16)] = 1.0 / (1.0 + jnp.exp(-u1))
            return carry

        lax.fori_loop(0, 8, fin, 0)
        pltpu.sync_copy(ob0, o0.at[pl.ds(s * 128, 128)])
        pltpu.sync_copy(ob1, o1.at[pl.ds(s * 128, 128)])


_lay1_call = functools.partial(
    pl.kernel,
    out_type=(jax.ShapeDtypeStruct((SUBP,), jnp.float32),
              jax.ShapeDtypeStruct((SUBP,), jnp.float32)),
    mesh=_MESH,
    scratch_types=[
        pltpu.VMEM((RPW1 // 2, KB), jnp.int32),
        pltpu.VMEM((RPW1 // 2, KB), jnp.int32),
        pltpu.VMEM((RPW1, KB), jnp.int32),
        pltpu.VMEM((RPW1 * KB + KB,), jnp.int32),
        pltpu.VMEM((RPW1 * KB + KB,), jnp.int32),
        pltpu.VMEM((KB,), jnp.float32),
        pltpu.VMEM((KB,), jnp.float32),
        pltpu.VMEM((HIST,), jnp.float32),
        pltpu.VMEM((HIST,), jnp.float32),
        pltpu.VMEM((128,), jnp.float32),
        pltpu.VMEM((128,), jnp.float32),
        pltpu.VMEM((128,), jnp.float32),
        pltpu.VMEM((128,), jnp.float32),
        pltpu.VMEM((128,), jnp.float32),
        pltpu.VMEM((2 * 16,), jnp.float32),
        pltpu.VMEM_SHARED((SUBP,), jnp.float32),
        pltpu.VMEM_SHARED((SUBP,), jnp.float32),
    ],
    compiler_params=pltpu.CompilerParams(needs_layout_passes=False),
)(_lay1_body)


def kernel(features, uni_to_sub, sub_to_uni, num_elements, num_subsets,
           W_sub0, b_sub0, W_uni0, b_uni0, W_sub1, b_sub1, W_uni1, b_uni1):
    del num_elements, num_subsets, W_uni1, b_uni1
    npad = EP - E
    pad_src = (jnp.arange(npad, dtype=jnp.int32) * 97) % N
    pad_dst = N + (jnp.arange(npad, dtype=jnp.int32) % (HIST - N))
    pad = jnp.stack([pad_src, pad_dst], axis=0)
    u2s_r = jnp.concatenate([uni_to_sub, pad], axis=1).reshape(2, ROWS, KB)
    s2u_r = jnp.concatenate([sub_to_uni, pad], axis=1).reshape(2, ROWS, KB)
    zeros_z = jnp.zeros((ZCH,), jnp.float32)
    ones_k = jnp.ones((KB,), jnp.float32)
    zrow = jnp.zeros((128, D), jnp.float32)
    z128 = jnp.zeros((128,), jnp.float32)
    b1t = jnp.repeat(b_sub1.reshape(NNET), 16)

    deg1, deg2 = _deg_call(u2s_r, s2u_r, zeros_z, ones_k)
    y1, y2, dis1, dis2 = _prep_call(deg1, deg2, features)
    accS = _scatter_sub_call(y1, u2s_r, zrow)
    accU = _scatter_uni_call(y2, s2u_r, zrow)
    t0, t1 = _dense_call(accS, accU, features, dis1, dis2,
                         W_sub0, b_sub0, W_uni0, b_uni0, W_sub1)
    o0, o1 = _lay1_call(u2s_r, t0, t1, dis1, b1t, z128)
    return jnp.stack([o0[:NSUB], o1[:NSUB]], axis=1)
